# baseline (speedup 1.0000x reference)
import jax
import jax.numpy as jnp
from jax import lax
from jax.experimental import pallas as pl
from jax.experimental.pallas import tpu as pltpu
from jax.experimental.pallas import tpu_sc as plsc

_V = 1000000
_R = _V + 1
_D = 32
_B = 16384
_L = 16
_NW = 32
_BPW = _B // _NW
_K = 8


def _mf_kernel(user_ids_hbm, item_ids_hbm, ut_hbm, it_hbm,
               out_hbm, uidx_v, iidx_v, widx_v, urows_v, irows_v, out_v,
               sem_u, sem_i):
    wid = lax.axis_index("s") * 2 + lax.axis_index("c")
    base = wid * _BPW

    pltpu.sync_copy(user_ids_hbm.at[pl.ds(base, _BPW)], uidx_v)
    pltpu.sync_copy(item_ids_hbm.at[pl.ds(base, _BPW)], iidx_v)

    def fix(k, carry):
        u = uidx_v[pl.ds(k * _L, _L)]
        uidx_v[pl.ds(k * _L, _L)] = jnp.where((u >= 0) & (u < _V), u + 1, 0)
        i = iidx_v[pl.ds(k * _L, _L)]
        iidx_v[pl.ds(k * _L, _L)] = jnp.where((i >= 0) & (i < _V), i + 1, 0)
        return carry
    lax.fori_loop(0, _BPW // _L, fix, 0)

    def bidx(d, carry):
        def one(k, carry2):
            sl = pl.ds(k * _L, _L)
            widx_v[pl.ds(d * _BPW + k * _L, _L)] = uidx_v[sl] + d * _R
            widx_v[pl.ds((_D + d) * _BPW + k * _L, _L)] = iidx_v[sl] + d * _R
            return carry2
        return lax.fori_loop(0, _BPW // _L, one, carry)
    lax.fori_loop(0, _D, bidx, 0)

    def fetch(g, carry):
        cps = []
        for j in range(_K):
            d = g * _K + j
            cu = pltpu.make_async_copy(
                ut_hbm.at[widx_v.at[pl.ds(d * _BPW, _BPW)]],
                urows_v.at[pl.ds(d * _BPW, _BPW)], sem_u)
            ci = pltpu.make_async_copy(
                it_hbm.at[widx_v.at[pl.ds((_D + d) * _BPW, _BPW)]],
                irows_v.at[pl.ds(d * _BPW, _BPW)], sem_i)
            cu.start()
            ci.start()
            cps.append(cu)
            cps.append(ci)
        for cp in cps:
            cp.wait()
        return carry
    lax.fori_loop(0, _D // _K, fetch, 0)

    def dot_block(rb, carry):
        def step(d, acc):
            sl = pl.ds(d * _BPW + rb * _L, _L)
            return acc + urows_v[sl] * irows_v[sl]
        acc = lax.fori_loop(0, _D, step, jnp.zeros((_L,), jnp.float32))
        out_v[pl.ds(rb * _L, _L)] = acc
        return carry
    lax.fori_loop(0, _BPW // _L, dot_block, 0)

    pltpu.sync_copy(out_v, out_hbm.at[pl.ds(base, _BPW)])


@jax.jit
def kernel(user_ids, item_ids, user_table, item_table):
    mesh = plsc.VectorSubcoreMesh(core_axis_name="c", subcore_axis_name="s")
    run = pl.kernel(
        _mf_kernel,
        out_type=jax.ShapeDtypeStruct((_B,), jnp.float32),
        mesh=mesh,
        compiler_params=pltpu.CompilerParams(needs_layout_passes=False),
        scratch_types=[
            pltpu.VMEM((_BPW,), jnp.int32),
            pltpu.VMEM((_BPW,), jnp.int32),
            pltpu.VMEM((2 * _D * _BPW,), jnp.int32),
            pltpu.VMEM((_D * _BPW,), jnp.float32),
            pltpu.VMEM((_D * _BPW,), jnp.float32),
            pltpu.VMEM((_BPW,), jnp.float32),
            pltpu.SemaphoreType.DMA,
            pltpu.SemaphoreType.DMA,
        ],
    )
    return run(user_ids, item_ids,
               user_table.T.ravel(), item_table.T.ravel())


# speedup vs baseline: 17.1054x; 17.1054x over previous
import jax
import jax.numpy as jnp
from jax import lax
from jax.experimental import pallas as pl
from jax.experimental.pallas import tpu as pltpu
from jax.experimental.pallas import tpu_sc as plsc

_V = 1000000
_R = _V + 1
_D = 32
_B = 16384
_L = 16
_NW = 32
_BPW = _B // _NW
_CW = 512
_NCHUNK = _R // _CW
_TAIL0 = _NCHUNK * _CW
_TAIL = _R - _TAIL0
_CHUNK_WORDS = _D * _CW
_FLAT = _NCHUNK * _CHUNK_WORDS
_SLOTS = _NCHUNK // _NW
_K = 8


def _detile_kernel(ut_hbm, it_hbm, uflat_hbm, iflat_hbm,
                   ubuf, ibuf, sem_in, sem_out):
    wid = lax.axis_index("s") * 2 + lax.axis_index("c")

    def do_chunk(chunk, buf_slot):
        col0 = pl.multiple_of(chunk * _CW, _CW)
        base = chunk * _CHUNK_WORDS
        cps = [
            pltpu.make_async_copy(
                ut_hbm.at[:, pl.ds(col0, _CW)], ubuf.at[buf_slot], sem_in),
            pltpu.make_async_copy(
                it_hbm.at[:, pl.ds(col0, _CW)], ibuf.at[buf_slot], sem_in),
        ]
        for cp in cps:
            cp.start()
        for cp in cps:
            cp.wait()
        cps = []
        for d in range(_D):
            cps.append(pltpu.make_async_copy(
                ubuf.at[buf_slot, d],
                uflat_hbm.at[pl.ds(base + d * _CW, _CW)], sem_out))
            cps.append(pltpu.make_async_copy(
                ibuf.at[buf_slot, d],
                iflat_hbm.at[pl.ds(base + d * _CW, _CW)], sem_out))
        for cp in cps:
            cp.start()
        for cp in cps:
            cp.wait()

    def body(slot, carry):
        do_chunk(slot * _NW + wid, 0)
        return carry
    lax.fori_loop(0, _SLOTS, body, 0)

    @pl.when(wid == 0)
    def _():
        do_chunk(_NCHUNK - 1, 0)


def _gather_dot_kernel(user_ids_hbm, item_ids_hbm, uflat_hbm, iflat_hbm,
                       utail_hbm, itail_hbm, out_hbm,
                       uidx_v, iidx_v, widx_v, tidx_v, urows_v, irows_v,
                       utail_v, itail_v, out_v, sem_u, sem_i):
    wid = lax.axis_index("s") * 2 + lax.axis_index("c")
    base = wid * _BPW

    pltpu.sync_copy(user_ids_hbm.at[pl.ds(base, _BPW)], uidx_v)
    pltpu.sync_copy(item_ids_hbm.at[pl.ds(base, _BPW)], iidx_v)
    pltpu.sync_copy(utail_hbm, utail_v)
    pltpu.sync_copy(itail_hbm, itail_v)

    def fix(k, carry):
        sl = pl.ds(k * _L, _L)
        u = uidx_v[sl]
        u = jnp.where((u >= 0) & (u < _V), u + 1, 0)
        ut_tail = u >= _TAIL0
        tidx_v[sl] = jnp.where(ut_tail, u - _TAIL0 + 1, 0)
        uidx_v[sl] = jnp.where(
            ut_tail, 0, (u >> 9) * _CHUNK_WORDS + (u & (_CW - 1)))
        i = iidx_v[sl]
        i = jnp.where((i >= 0) & (i < _V), i + 1, 0)
        it_tail = i >= _TAIL0
        tidx_v[pl.ds(_BPW + k * _L, _L)] = jnp.where(it_tail, i - _TAIL0 + 1, 0)
        iidx_v[sl] = jnp.where(
            it_tail, 0, (i >> 9) * _CHUNK_WORDS + (i & (_CW - 1)))
        return carry
    lax.fori_loop(0, _BPW // _L, fix, 0)

    def bidx(d, carry):
        def one(k, carry2):
            sl = pl.ds(k * _L, _L)
            widx_v[pl.ds(d * _BPW + k * _L, _L)] = uidx_v[sl] + d * _CW
            widx_v[pl.ds((_D + d) * _BPW + k * _L, _L)] = iidx_v[sl] + d * _CW
            return carry2
        return lax.fori_loop(0, _BPW // _L, one, carry)
    lax.fori_loop(0, _D, bidx, 0)

    def fetch(g, carry):
        cps = []
        for j in range(_K):
            d = g * _K + j
            cu = pltpu.make_async_copy(
                uflat_hbm.at[widx_v.at[pl.ds(d * _BPW, _BPW)]],
                urows_v.at[pl.ds(d * _BPW, _BPW)], sem_u)
            ci = pltpu.make_async_copy(
                iflat_hbm.at[widx_v.at[pl.ds((_D + d) * _BPW, _BPW)]],
                irows_v.at[pl.ds(d * _BPW, _BPW)], sem_i)
            cu.start()
            ci.start()
            cps.append(cu)
            cps.append(ci)
        for cp in cps:
            cp.wait()
        return carry
    lax.fori_loop(0, _D // _K, fetch, 0)

    def dot_block(rb, carry):
        sl = pl.ds(rb * _L, _L)
        ut_off = tidx_v[sl]
        it_off = tidx_v[pl.ds(_BPW + rb * _L, _L)]
        ut_mask = ut_off > 0
        it_mask = it_off > 0

        def step(d, acc):
            msl = pl.ds(d * _BPW + rb * _L, _L)
            u = urows_v[msl]
            i = irows_v[msl]
            ut = plsc.load_gather(utail_v, [ut_off - 1 + d * 128])
            it_ = plsc.load_gather(itail_v, [it_off - 1 + d * 128])
            u = jnp.where(ut_mask, ut, u)
            i = jnp.where(it_mask, it_, i)
            return acc + u * i
        acc = lax.fori_loop(0, _D, step, jnp.zeros((_L,), jnp.float32))
        out_v[sl] = acc
        return carry
    lax.fori_loop(0, _BPW // _L, dot_block, 0)

    pltpu.sync_copy(out_v, out_hbm.at[pl.ds(base, _BPW)])


@jax.jit
def kernel(user_ids, item_ids, user_table, item_table):
    mesh = plsc.VectorSubcoreMesh(core_axis_name="c", subcore_axis_name="s")
    detile = pl.kernel(
        _detile_kernel,
        out_type=(
            jax.ShapeDtypeStruct((_FLAT,), jnp.float32),
            jax.ShapeDtypeStruct((_FLAT,), jnp.float32),
        ),
        mesh=mesh,
        compiler_params=pltpu.CompilerParams(
            needs_layout_passes=False, use_tc_tiling_on_sc=True),
        scratch_types=[
            pltpu.VMEM((1, _D, _CW), jnp.float32),
            pltpu.VMEM((1, _D, _CW), jnp.float32),
            pltpu.SemaphoreType.DMA,
            pltpu.SemaphoreType.DMA,
        ],
    )
    ut_t = user_table.T
    it_t = item_table.T
    uflat, iflat = detile(ut_t, it_t)

    utail = jnp.pad(lax.slice(ut_t, (0, _TAIL0), (_D, _R)),
                    ((0, 0), (0, 128 - _TAIL))).ravel()
    itail = jnp.pad(lax.slice(it_t, (0, _TAIL0), (_D, _R)),
                    ((0, 0), (0, 128 - _TAIL))).ravel()

    gather_dot = pl.kernel(
        _gather_dot_kernel,
        out_type=jax.ShapeDtypeStruct((_B,), jnp.float32),
        mesh=mesh,
        compiler_params=pltpu.CompilerParams(needs_layout_passes=False),
        scratch_types=[
            pltpu.VMEM((_BPW,), jnp.int32),
            pltpu.VMEM((_BPW,), jnp.int32),
            pltpu.VMEM((2 * _D * _BPW,), jnp.int32),
            pltpu.VMEM((2 * _BPW,), jnp.int32),
            pltpu.VMEM((_D * _BPW,), jnp.float32),
            pltpu.VMEM((_D * _BPW,), jnp.float32),
            pltpu.VMEM((_D * 128,), jnp.float32),
            pltpu.VMEM((_D * 128,), jnp.float32),
            pltpu.VMEM((_BPW,), jnp.float32),
            pltpu.SemaphoreType.DMA,
            pltpu.SemaphoreType.DMA,
        ],
    )
    return gather_dot(user_ids, item_ids, uflat, iflat, utail, itail)


# speedup vs baseline: 19.9391x; 1.1657x over previous
import jax
import jax.numpy as jnp
from jax import lax
from jax.experimental import pallas as pl
from jax.experimental.pallas import tpu as pltpu
from jax.experimental.pallas import tpu_sc as plsc

_V = 1000000
_R = _V + 1
_D = 32
_B = 16384
_L = 16
_NW = 32
_BPW = _B // _NW
_CW = 512
_NCHUNK = _R // _CW
_TAIL0 = _NCHUNK * _CW
_TAIL = _R - _TAIL0
_CHUNK_WORDS = _D * _CW
_FLAT = _NCHUNK * _CHUNK_WORDS
_SLOTS = _NCHUNK // _NW
_K = 16


def _detile_kernel(ut_hbm, it_hbm, uflat_hbm, iflat_hbm,
                   ubuf, ibuf, sem_in, sem_out):
    wid = lax.axis_index("s") * 2 + lax.axis_index("c")

    def in_copies(slot, s):
        col0 = pl.multiple_of((slot * _NW + wid) * _CW, _CW)
        return [
            pltpu.make_async_copy(
                ut_hbm.at[:, pl.ds(col0, _CW)], ubuf.at[s], sem_in),
            pltpu.make_async_copy(
                it_hbm.at[:, pl.ds(col0, _CW)], ibuf.at[s], sem_in),
        ]

    def out_copies(slot, s):
        base = (slot * _NW + wid) * _CHUNK_WORDS
        cps = []
        for d in range(_D):
            cps.append(pltpu.make_async_copy(
                ubuf.at[s, d],
                uflat_hbm.at[pl.ds(base + d * _CW, _CW)], sem_out))
            cps.append(pltpu.make_async_copy(
                ibuf.at[s, d],
                iflat_hbm.at[pl.ds(base + d * _CW, _CW)], sem_out))
        return cps

    for cp in in_copies(0, 0):
        cp.start()
    for cp in in_copies(1, 1):
        cp.start()

    def body(i, carry):
        s = lax.rem(i, 3)
        for cp in in_copies(i, s):
            cp.wait()
        for cp in out_copies(i, s):
            cp.start()

        @pl.when(i >= 1)
        def _():
            for cp in out_copies(i - 1, lax.rem(i + 2, 3)):
                cp.wait()

        @pl.when(i + 2 < _SLOTS)
        def _():
            for cp in in_copies(i + 2, lax.rem(i + 2, 3)):
                cp.start()
        return carry
    lax.fori_loop(0, _SLOTS, body, 0)
    for cp in out_copies(_SLOTS - 1, lax.rem(_SLOTS - 1, 3)):
        cp.wait()

    @pl.when(wid == 0)
    def _():
        for cp in in_copies(_SLOTS, 0):
            cp.start()
        for cp in in_copies(_SLOTS, 0):
            cp.wait()
        for cp in out_copies(_SLOTS, 0):
            cp.start()
        for cp in out_copies(_SLOTS, 0):
            cp.wait()


def _gather_dot_kernel(user_ids_hbm, item_ids_hbm, uflat_hbm, iflat_hbm,
                       utail_hbm, itail_hbm, out_hbm,
                       uidx_v, iidx_v, widx_v, tidx_v, urows_v, irows_v,
                       utail_v, itail_v, out_v, sem_u, sem_i):
    wid = lax.axis_index("s") * 2 + lax.axis_index("c")
    base = wid * _BPW

    pltpu.sync_copy(user_ids_hbm.at[pl.ds(base, _BPW)], uidx_v)
    pltpu.sync_copy(item_ids_hbm.at[pl.ds(base, _BPW)], iidx_v)
    pltpu.sync_copy(utail_hbm, utail_v)
    pltpu.sync_copy(itail_hbm, itail_v)

    def fix(k, carry):
        sl = pl.ds(k * _L, _L)
        u = uidx_v[sl]
        u = jnp.where((u >= 0) & (u < _V), u + 1, 0)
        ut_tail = u >= _TAIL0
        tidx_v[sl] = jnp.where(ut_tail, u - _TAIL0 + 1, 0)
        uidx_v[sl] = jnp.where(
            ut_tail, 0, (u >> 9) * _CHUNK_WORDS + (u & (_CW - 1)))
        i = iidx_v[sl]
        i = jnp.where((i >= 0) & (i < _V), i + 1, 0)
        it_tail = i >= _TAIL0
        tidx_v[pl.ds(_BPW + k * _L, _L)] = jnp.where(it_tail, i - _TAIL0 + 1, 0)
        iidx_v[sl] = jnp.where(
            it_tail, 0, (i >> 9) * _CHUNK_WORDS + (i & (_CW - 1)))
        return carry
    lax.fori_loop(0, _BPW // _L, fix, 0)

    def bidx(d, carry):
        def one(k, carry2):
            sl = pl.ds(k * _L, _L)
            widx_v[pl.ds(d * _BPW + k * _L, _L)] = uidx_v[sl] + d * _CW
            widx_v[pl.ds((_D + d) * _BPW + k * _L, _L)] = iidx_v[sl] + d * _CW
            return carry2
        return lax.fori_loop(0, _BPW // _L, one, carry)
    lax.fori_loop(0, _D, bidx, 0)

    def fetch(g, carry):
        cps = []
        for j in range(_K):
            d = g * _K + j
            cu = pltpu.make_async_copy(
                uflat_hbm.at[widx_v.at[pl.ds(d * _BPW, _BPW)]],
                urows_v.at[pl.ds(d * _BPW, _BPW)], sem_u)
            ci = pltpu.make_async_copy(
                iflat_hbm.at[widx_v.at[pl.ds((_D + d) * _BPW, _BPW)]],
                irows_v.at[pl.ds(d * _BPW, _BPW)], sem_i)
            cu.start()
            ci.start()
            cps.append(cu)
            cps.append(ci)
        for cp in cps:
            cp.wait()
        return carry
    lax.fori_loop(0, _D // _K, fetch, 0)

    def dot_block(rb, carry):
        sl = pl.ds(rb * _L, _L)
        ut_off = tidx_v[sl]
        it_off = tidx_v[pl.ds(_BPW + rb * _L, _L)]
        ut_mask = ut_off > 0
        it_mask = it_off > 0

        def step(d, acc):
            msl = pl.ds(d * _BPW + rb * _L, _L)
            u = urows_v[msl]
            i = irows_v[msl]
            ut = plsc.load_gather(utail_v, [ut_off - 1 + d * 128])
            it_ = plsc.load_gather(itail_v, [it_off - 1 + d * 128])
            u = jnp.where(ut_mask, ut, u)
            i = jnp.where(it_mask, it_, i)
            return acc + u * i
        acc = lax.fori_loop(0, _D, step, jnp.zeros((_L,), jnp.float32))
        out_v[sl] = acc
        return carry
    lax.fori_loop(0, _BPW // _L, dot_block, 0)

    pltpu.sync_copy(out_v, out_hbm.at[pl.ds(base, _BPW)])


@jax.jit
def kernel(user_ids, item_ids, user_table, item_table):
    mesh = plsc.VectorSubcoreMesh(core_axis_name="c", subcore_axis_name="s")
    detile = pl.kernel(
        _detile_kernel,
        out_type=(
            jax.ShapeDtypeStruct((_FLAT,), jnp.float32),
            jax.ShapeDtypeStruct((_FLAT,), jnp.float32),
        ),
        mesh=mesh,
        compiler_params=pltpu.CompilerParams(
            needs_layout_passes=False, use_tc_tiling_on_sc=True),
        scratch_types=[
            pltpu.VMEM((3, _D, _CW), jnp.float32),
            pltpu.VMEM((3, _D, _CW), jnp.float32),
            pltpu.SemaphoreType.DMA,
            pltpu.SemaphoreType.DMA,
        ],
    )
    ut_t = user_table.T
    it_t = item_table.T
    uflat, iflat = detile(ut_t, it_t)

    utail = jnp.pad(lax.slice(ut_t, (0, _TAIL0), (_D, _R)),
                    ((0, 0), (0, 128 - _TAIL))).ravel()
    itail = jnp.pad(lax.slice(it_t, (0, _TAIL0), (_D, _R)),
                    ((0, 0), (0, 128 - _TAIL))).ravel()

    gather_dot = pl.kernel(
        _gather_dot_kernel,
        out_type=jax.ShapeDtypeStruct((_B,), jnp.float32),
        mesh=mesh,
        compiler_params=pltpu.CompilerParams(needs_layout_passes=False),
        scratch_types=[
            pltpu.VMEM((_BPW,), jnp.int32),
            pltpu.VMEM((_BPW,), jnp.int32),
            pltpu.VMEM((2 * _D * _BPW,), jnp.int32),
            pltpu.VMEM((2 * _BPW,), jnp.int32),
            pltpu.VMEM((_D * _BPW,), jnp.float32),
            pltpu.VMEM((_D * _BPW,), jnp.float32),
            pltpu.VMEM((_D * 128,), jnp.float32),
            pltpu.VMEM((_D * 128,), jnp.float32),
            pltpu.VMEM((_BPW,), jnp.float32),
            pltpu.SemaphoreType.DMA,
            pltpu.SemaphoreType.DMA,
        ],
    )
    return gather_dot(user_ids, item_ids, uflat, iflat, utail, itail)


# speedup vs baseline: 20.0673x; 1.0064x over previous
import jax
import jax.numpy as jnp
from jax import lax
from jax.experimental import pallas as pl
from jax.experimental.pallas import tpu as pltpu
from jax.experimental.pallas import tpu_sc as plsc

_V = 1000000
_R = _V + 1
_D = 32
_B = 16384
_L = 16
_NW = 32
_BPW = _B // _NW
_CW = 512
_NCHUNK = _R // _CW
_TAIL0 = _NCHUNK * _CW
_TAIL = _R - _TAIL0
_CHUNK_WORDS = _D * _CW
_FLAT = _NCHUNK * _CHUNK_WORDS
_SLOTS = _NCHUNK // _NW
_K = 16


def _detile_kernel(ut_hbm, it_hbm, uflat_hbm, iflat_hbm,
                   ubuf, ibuf, sem_in, sem_out):
    wid = lax.axis_index("s") * 2 + lax.axis_index("c")

    def in_copies(slot, s):
        col0 = pl.multiple_of((slot * _NW + wid) * _CW, _CW)
        return [
            pltpu.make_async_copy(
                ut_hbm.at[:, pl.ds(col0, _CW)], ubuf.at[s], sem_in),
            pltpu.make_async_copy(
                it_hbm.at[:, pl.ds(col0, _CW)], ibuf.at[s], sem_in),
        ]

    def out_copies(slot, s):
        base = (slot * _NW + wid) * _CHUNK_WORDS
        cps = []
        for d in range(_D):
            cps.append(pltpu.make_async_copy(
                ubuf.at[s, d],
                uflat_hbm.at[pl.ds(base + d * _CW, _CW)], sem_out))
            cps.append(pltpu.make_async_copy(
                ibuf.at[s, d],
                iflat_hbm.at[pl.ds(base + d * _CW, _CW)], sem_out))
        return cps

    for cp in in_copies(0, 0):
        cp.start()
    for cp in in_copies(1, 1):
        cp.start()

    def body(i, carry):
        s = lax.rem(i, 3)
        for cp in in_copies(i, s):
            cp.wait()
        for cp in out_copies(i, s):
            cp.start()

        @pl.when(i >= 1)
        def _():
            for cp in out_copies(i - 1, lax.rem(i + 2, 3)):
                cp.wait()

        @pl.when(i + 2 < _SLOTS)
        def _():
            for cp in in_copies(i + 2, lax.rem(i + 2, 3)):
                cp.start()
        return carry
    lax.fori_loop(0, _SLOTS, body, 0)
    for cp in out_copies(_SLOTS - 1, lax.rem(_SLOTS - 1, 3)):
        cp.wait()

    @pl.when(wid == 0)
    def _():
        for cp in in_copies(_SLOTS, 0):
            cp.start()
        for cp in in_copies(_SLOTS, 0):
            cp.wait()
        for cp in out_copies(_SLOTS, 0):
            cp.start()
        for cp in out_copies(_SLOTS, 0):
            cp.wait()


def _gather_dot_kernel(user_ids_hbm, item_ids_hbm, uflat_hbm, iflat_hbm,
                       utail_hbm, itail_hbm, out_hbm,
                       uidx_v, iidx_v, widx_v, tidx_v, urows_v, irows_v,
                       utail_v, itail_v, out_v, sem_u, sem_i):
    wid = lax.axis_index("s") * 2 + lax.axis_index("c")
    base = wid * _BPW

    pltpu.sync_copy(user_ids_hbm.at[pl.ds(base, _BPW)], uidx_v)
    pltpu.sync_copy(item_ids_hbm.at[pl.ds(base, _BPW)], iidx_v)
    pltpu.sync_copy(utail_hbm, utail_v)
    pltpu.sync_copy(itail_hbm, itail_v)

    def fix(k, carry):
        sl = pl.ds(k * _L, _L)
        u = uidx_v[sl]
        u = jnp.where((u >= 0) & (u < _V), u + 1, 0)
        ut_tail = u >= _TAIL0
        tidx_v[sl] = jnp.where(ut_tail, u - _TAIL0 + 1, 0)
        uidx_v[sl] = jnp.where(
            ut_tail, 0, (u >> 9) * _CHUNK_WORDS + (u & (_CW - 1)))
        i = iidx_v[sl]
        i = jnp.where((i >= 0) & (i < _V), i + 1, 0)
        it_tail = i >= _TAIL0
        tidx_v[pl.ds(_BPW + k * _L, _L)] = jnp.where(it_tail, i - _TAIL0 + 1, 0)
        iidx_v[sl] = jnp.where(
            it_tail, 0, (i >> 9) * _CHUNK_WORDS + (i & (_CW - 1)))
        return carry
    lax.fori_loop(0, _BPW // _L, fix, 0)

    def bidx(d, carry):
        def one(k, carry2):
            sl = pl.ds(k * _L, _L)
            widx_v[pl.ds(d * _BPW + k * _L, _L)] = uidx_v[sl] + d * _CW
            widx_v[pl.ds((_D + d) * _BPW + k * _L, _L)] = iidx_v[sl] + d * _CW
            return carry2
        return lax.fori_loop(0, _BPW // _L, one, carry)
    lax.fori_loop(0, _D, bidx, 0)

    def round_copies(g):
        cps = []
        for j in range(_K):
            d = g * _K + j
            cps.append(pltpu.make_async_copy(
                uflat_hbm.at[widx_v.at[pl.ds(d * _BPW, _BPW)]],
                urows_v.at[pl.ds(d * _BPW, _BPW)], sem_u))
            cps.append(pltpu.make_async_copy(
                iflat_hbm.at[widx_v.at[pl.ds((_D + d) * _BPW, _BPW)]],
                irows_v.at[pl.ds(d * _BPW, _BPW)], sem_i))
        return cps

    _NROUND = _D // _K
    for cp in round_copies(0):
        cp.start()
    for g in range(_NROUND):
        if g + 1 < _NROUND:
            for cp in round_copies(g + 1):
                cp.start()
        for cp in round_copies(g):
            cp.wait()

        def dot_block(rb, carry, g=g):
            sl = pl.ds(rb * _L, _L)
            ut_off = tidx_v[sl]
            it_off = tidx_v[pl.ds(_BPW + rb * _L, _L)]
            ut_mask = ut_off > 0
            it_mask = it_off > 0
            acc = jnp.zeros((_L,), jnp.float32) if g == 0 else out_v[sl]
            for d in range(g * _K, (g + 1) * _K):
                msl = pl.ds(d * _BPW + rb * _L, _L)
                u = urows_v[msl]
                i = irows_v[msl]
                ut = plsc.load_gather(utail_v, [ut_off - 1 + d * 128])
                it_ = plsc.load_gather(itail_v, [it_off - 1 + d * 128])
                u = jnp.where(ut_mask, ut, u)
                i = jnp.where(it_mask, it_, i)
                acc = acc + u * i
            out_v[sl] = acc
            return carry
        lax.fori_loop(0, _BPW // _L, dot_block, 0)

    pltpu.sync_copy(out_v, out_hbm.at[pl.ds(base, _BPW)])


@jax.jit
def kernel(user_ids, item_ids, user_table, item_table):
    mesh = plsc.VectorSubcoreMesh(core_axis_name="c", subcore_axis_name="s")
    detile = pl.kernel(
        _detile_kernel,
        out_type=(
            jax.ShapeDtypeStruct((_FLAT,), jnp.float32),
            jax.ShapeDtypeStruct((_FLAT,), jnp.float32),
        ),
        mesh=mesh,
        compiler_params=pltpu.CompilerParams(
            needs_layout_passes=False, use_tc_tiling_on_sc=True),
        scratch_types=[
            pltpu.VMEM((3, _D, _CW), jnp.float32),
            pltpu.VMEM((3, _D, _CW), jnp.float32),
            pltpu.SemaphoreType.DMA,
            pltpu.SemaphoreType.DMA,
        ],
    )
    ut_t = user_table.T
    it_t = item_table.T
    uflat, iflat = detile(ut_t, it_t)

    utail = jnp.pad(lax.slice(ut_t, (0, _TAIL0), (_D, _R)),
                    ((0, 0), (0, 128 - _TAIL))).ravel()
    itail = jnp.pad(lax.slice(it_t, (0, _TAIL0), (_D, _R)),
                    ((0, 0), (0, 128 - _TAIL))).ravel()

    gather_dot = pl.kernel(
        _gather_dot_kernel,
        out_type=jax.ShapeDtypeStruct((_B,), jnp.float32),
        mesh=mesh,
        compiler_params=pltpu.CompilerParams(needs_layout_passes=False),
        scratch_types=[
            pltpu.VMEM((_BPW,), jnp.int32),
            pltpu.VMEM((_BPW,), jnp.int32),
            pltpu.VMEM((2 * _D * _BPW,), jnp.int32),
            pltpu.VMEM((2 * _BPW,), jnp.int32),
            pltpu.VMEM((_D * _BPW,), jnp.float32),
            pltpu.VMEM((_D * _BPW,), jnp.float32),
            pltpu.VMEM((_D * 128,), jnp.float32),
            pltpu.VMEM((_D * 128,), jnp.float32),
            pltpu.VMEM((_BPW,), jnp.float32),
            pltpu.SemaphoreType.DMA,
            pltpu.SemaphoreType.DMA,
        ],
    )
    return gather_dot(user_ids, item_ids, uflat, iflat, utail, itail)
